# Initial kernel scaffold; baseline (speedup 1.0000x reference)
import functools

import jax
import jax.numpy as jnp
from jax import lax
from jax.experimental import pallas as pl
from jax.experimental.pallas import tpu as pltpu, tpu_sc as plsc

_L = 16


def _sc_encode(bin_f32, iin_i32, gt_f32, B, Np, G):
    info = plsc.get_sparse_core_info()
    NC, NS = info.num_cores, info.num_subcores
    assert NC * NS == B, (NC, NS, B)
    n_chunks = Np // _L
    mesh = plsc.VectorSubcoreMesh(core_axis_name="c", subcore_axis_name="s")

    @functools.partial(
        pl.kernel,
        out_type=jax.ShapeDtypeStruct((B, 5 * Np), jnp.float32),
        mesh=mesh,
        scratch_types=[
            pltpu.VMEM((4 * Np,), jnp.float32),
            pltpu.VMEM((2 * Np,), jnp.int32),
            pltpu.VMEM((6 * G,), jnp.float32),
            pltpu.VMEM((5 * Np,), jnp.float32),
        ],
    )
    def body(bin_hbm, iin_hbm, gt_hbm, out_hbm, bin_v, iin_v, gt_v, out_v):
        w = lax.axis_index("s") * NC + lax.axis_index("c")
        pltpu.sync_copy(bin_hbm.at[w], bin_v)
        pltpu.sync_copy(iin_hbm.at[w], iin_v)
        pltpu.sync_copy(gt_hbm.at[w], gt_v)

        def step(c, carry):
            s = c * _L
            gid = iin_v[pl.ds(s, _L)]
            flag = iin_v[pl.ds(Np + s, _L)]
            bx1 = bin_v[pl.ds(s, _L)]
            by1 = bin_v[pl.ds(Np + s, _L)]
            bx2 = bin_v[pl.ds(2 * Np + s, _L)]
            by2 = bin_v[pl.ds(3 * Np + s, _L)]
            cls = plsc.load_gather(gt_v, [gid])
            gx1 = plsc.load_gather(gt_v, [gid + G])
            gy1 = plsc.load_gather(gt_v, [gid + 2 * G])
            gx2 = plsc.load_gather(gt_v, [gid + 3 * G])
            gy2 = plsc.load_gather(gt_v, [gid + 4 * G])
            gcl = plsc.load_gather(gt_v, [gid + 5 * G])

            pos = flag > 0
            regm0 = jnp.logical_and(pos, cls > 0.0)
            g = jnp.where(flag == 0, 0.0, gcl)
            g = jnp.where(flag < 0, -1.0, g)
            g = jnp.where(cls == 0.0, -1.0, g)
            clsm = g >= 0.0
            dx = gx1 - gx2
            dy = gy1 - gy2
            dxz = dx == 0.0
            dxs = jnp.where(dxz, 1.0, dx)
            slope = dy / dxs
            cy1 = jnp.where(dxz, 0.0, slope * (bx1 - gx2) + gy2)
            cy2 = jnp.where(dxz, 0.0, slope * (bx2 - gx2) + gy2)
            bw = bx2 - bx1
            bh = by2 - by1
            m = jnp.logical_and(jnp.logical_and(bw > 0.0, bh > 0.0),
                                jnp.logical_not(dxz))
            r1 = (cy1 - by2) / bh
            r2 = (cy2 - by2) / bh
            regm = jnp.logical_and(regm0, m)

            out_v[pl.ds(s, _L)] = g
            out_v[pl.ds(Np + s, _L)] = jnp.where(clsm, 1.0, 0.0)
            out_v[pl.ds(2 * Np + s, _L)] = r1
            out_v[pl.ds(3 * Np + s, _L)] = r2
            out_v[pl.ds(4 * Np + s, _L)] = jnp.where(regm, 1.0, 0.0)
            return carry

        lax.fori_loop(0, n_chunks, step, None)
        pltpu.sync_copy(out_v, out_hbm.at[w])

    return body(bin_f32, iin_i32, gt_f32)


@jax.jit
def kernel(boxes, gt_boxes, gt_flanks, match_pos_flag, match_gt_id):
    B, N, _ = boxes.shape
    G = gt_boxes.shape[1]
    Np = ((N + _L - 1) // _L) * _L
    pad = Np - N

    bt = jnp.pad(jnp.transpose(boxes, (0, 2, 1)), ((0, 0), (0, 0), (0, pad)))
    bin_f32 = bt.reshape(B, 4 * Np)
    gi = jnp.pad(match_gt_id, ((0, 0), (0, pad)))
    fl = jnp.pad(match_pos_flag, ((0, 0), (0, pad)))
    iin_i32 = jnp.concatenate([gi, fl], axis=1)
    gt_f32 = jnp.stack(
        [gt_boxes[:, :, 4], gt_flanks[:, :, 0], gt_flanks[:, :, 1],
         gt_flanks[:, :, 2], gt_flanks[:, :, 3], gt_flanks[:, :, 8]],
        axis=1).reshape(B, 6 * G)

    out = _sc_encode(bin_f32, iin_i32, gt_f32, B, Np, G)
    o = out.reshape(B, 5, Np)[:, :, :N]
    gdls_cls = o[:, 0][..., None]
    cls_label_mask = (o[:, 1] != 0.0)[..., None]
    reg_label = jnp.stack([o[:, 2], o[:, 3]], axis=-1)
    reg_label_mask = jnp.broadcast_to((o[:, 4] != 0.0)[..., None],
                                      reg_label.shape)
    return gdls_cls, cls_label_mask, reg_label, reg_label_mask


# speedup vs baseline: 18.2356x; 18.2356x over previous
import functools

import jax
import jax.numpy as jnp
from jax import lax
from jax.experimental import pallas as pl
from jax.experimental.pallas import tpu as pltpu, tpu_sc as plsc

_L = 16


def _sc_encode(bin_f32, iin_i32, gt_f32, B, Np, G):
    info = plsc.get_sparse_core_info()
    NC, NS = info.num_cores, info.num_subcores
    assert NC * NS == B, (NC, NS, B)
    n_chunks = Np // _L
    mesh = plsc.VectorSubcoreMesh(core_axis_name="c", subcore_axis_name="s")

    @functools.partial(
        pl.kernel,
        out_type=jax.ShapeDtypeStruct((B, 5 * Np), jnp.float32),
        mesh=mesh,
        compiler_params=pltpu.CompilerParams(needs_layout_passes=False),
        scratch_types=[
            pltpu.VMEM((4 * Np,), jnp.float32),
            pltpu.VMEM((2 * Np,), jnp.int32),
            pltpu.VMEM((6 * G,), jnp.float32),
            pltpu.VMEM((5 * Np,), jnp.float32),
        ],
    )
    def body(bin_hbm, iin_hbm, gt_hbm, out_hbm, bin_v, iin_v, gt_v, out_v):
        w = lax.axis_index("s") * NC + lax.axis_index("c")
        pltpu.sync_copy(bin_hbm.at[w], bin_v)
        pltpu.sync_copy(iin_hbm.at[w], iin_v)
        pltpu.sync_copy(gt_hbm.at[w], gt_v)

        def step(c, carry):
            s = c * _L
            gid = iin_v[pl.ds(s, _L)]
            flag = iin_v[pl.ds(Np + s, _L)]
            bx1 = bin_v[pl.ds(s, _L)]
            by1 = bin_v[pl.ds(Np + s, _L)]
            bx2 = bin_v[pl.ds(2 * Np + s, _L)]
            by2 = bin_v[pl.ds(3 * Np + s, _L)]
            cls = plsc.load_gather(gt_v, [gid])
            gx1 = plsc.load_gather(gt_v, [gid + G])
            gy1 = plsc.load_gather(gt_v, [gid + 2 * G])
            gx2 = plsc.load_gather(gt_v, [gid + 3 * G])
            gy2 = plsc.load_gather(gt_v, [gid + 4 * G])
            gcl = plsc.load_gather(gt_v, [gid + 5 * G])

            pos = flag > 0
            regm0 = jnp.logical_and(pos, cls > 0.0)
            g = jnp.where(flag == 0, 0.0, gcl)
            g = jnp.where(flag < 0, -1.0, g)
            g = jnp.where(cls == 0.0, -1.0, g)
            clsm = g >= 0.0
            dx = gx1 - gx2
            dy = gy1 - gy2
            dxz = dx == 0.0
            dxs = jnp.where(dxz, 1.0, dx)
            slope = dy / dxs
            cy1 = jnp.where(dxz, 0.0, slope * (bx1 - gx2) + gy2)
            cy2 = jnp.where(dxz, 0.0, slope * (bx2 - gx2) + gy2)
            bw = bx2 - bx1
            bh = by2 - by1
            m = jnp.logical_and(jnp.logical_and(bw > 0.0, bh > 0.0),
                                jnp.logical_not(dxz))
            r1 = (cy1 - by2) / bh
            r2 = (cy2 - by2) / bh
            regm = jnp.logical_and(regm0, m)

            out_v[pl.ds(s, _L)] = g
            out_v[pl.ds(Np + s, _L)] = jnp.where(clsm, 1.0, 0.0)
            out_v[pl.ds(2 * Np + s, _L)] = r1
            out_v[pl.ds(3 * Np + s, _L)] = r2
            out_v[pl.ds(4 * Np + s, _L)] = jnp.where(regm, 1.0, 0.0)
            return carry

        lax.fori_loop(0, n_chunks, step, None)
        pltpu.sync_copy(out_v, out_hbm.at[w])

    return body(bin_f32, iin_i32, gt_f32)


@jax.jit
def kernel(boxes, gt_boxes, gt_flanks, match_pos_flag, match_gt_id):
    B, N, _ = boxes.shape
    G = gt_boxes.shape[1]
    Np = ((N + _L - 1) // _L) * _L
    pad = Np - N

    bt = jnp.pad(jnp.transpose(boxes, (0, 2, 1)), ((0, 0), (0, 0), (0, pad)))
    bin_f32 = bt.reshape(B, 4 * Np)
    gi = jnp.pad(match_gt_id, ((0, 0), (0, pad)))
    fl = jnp.pad(match_pos_flag, ((0, 0), (0, pad)))
    iin_i32 = jnp.concatenate([gi, fl], axis=1)
    gt_f32 = jnp.stack(
        [gt_boxes[:, :, 4], gt_flanks[:, :, 0], gt_flanks[:, :, 1],
         gt_flanks[:, :, 2], gt_flanks[:, :, 3], gt_flanks[:, :, 8]],
        axis=1).reshape(B, 6 * G)

    out = _sc_encode(bin_f32, iin_i32, gt_f32, B, Np, G)
    o = out.reshape(B, 5, Np)[:, :, :N]
    gdls_cls = o[:, 0][..., None]
    cls_label_mask = (o[:, 1] != 0.0)[..., None]
    reg_label = jnp.stack([o[:, 2], o[:, 3]], axis=-1)
    reg_label_mask = jnp.broadcast_to((o[:, 4] != 0.0)[..., None],
                                      reg_label.shape)
    return gdls_cls, cls_label_mask, reg_label, reg_label_mask


# speedup vs baseline: 18.2367x; 1.0001x over previous
import functools

import jax
import jax.numpy as jnp
from jax import lax
from jax.experimental import pallas as pl
from jax.experimental.pallas import tpu as pltpu, tpu_sc as plsc

_L = 16


def _sc_encode(boxes2, gtb2, gtf2, gid2, flag2, B, N, G):
    info = plsc.get_sparse_core_info()
    NC, NS = info.num_cores, info.num_subcores
    assert NC * NS == B, (NC, NS, B)
    n_chunks = (N + _L - 1) // _L
    mesh = plsc.VectorSubcoreMesh(core_axis_name="c", subcore_axis_name="s")

    @functools.partial(
        pl.kernel,
        out_type=jax.ShapeDtypeStruct((B, 5 * N), jnp.float32),
        mesh=mesh,
        compiler_params=pltpu.CompilerParams(needs_layout_passes=False),
        scratch_types=[
            pltpu.VMEM((4 * N,), jnp.float32),
            pltpu.VMEM((N,), jnp.int32),
            pltpu.VMEM((N,), jnp.int32),
            pltpu.VMEM((5 * G,), jnp.float32),
            pltpu.VMEM((9 * G,), jnp.float32),
            pltpu.VMEM((5 * N,), jnp.float32),
            pltpu.SemaphoreType.DMA,
        ],
    )
    def body(boxes_hbm, gtb_hbm, gtf_hbm, gid_hbm, flag_hbm, out_hbm,
             box_v, gid_v, flag_v, gtb_v, gtf_v, out_v, sem):
        w = lax.axis_index("s") * NC + lax.axis_index("c")
        cps = [
            pltpu.async_copy(boxes_hbm.at[w], box_v, sem),
            pltpu.async_copy(gid_hbm.at[w], gid_v, sem),
            pltpu.async_copy(flag_hbm.at[w], flag_v, sem),
            pltpu.async_copy(gtb_hbm.at[w], gtb_v, sem),
            pltpu.async_copy(gtf_hbm.at[w], gtf_v, sem),
        ]
        for cp in cps:
            cp.wait()

        lane = lax.iota(jnp.int32, _L)

        def step(c, carry):
            s = jnp.minimum(c * _L, N - _L)
            gid = gid_v[pl.ds(s, _L)]
            flag = flag_v[pl.ds(s, _L)]
            bidx = (s + lane) * 4
            bx1 = plsc.load_gather(box_v, [bidx])
            by1 = plsc.load_gather(box_v, [bidx + 1])
            bx2 = plsc.load_gather(box_v, [bidx + 2])
            by2 = plsc.load_gather(box_v, [bidx + 3])
            cls = plsc.load_gather(gtb_v, [gid * 5 + 4])
            fidx = gid * 9
            gx1 = plsc.load_gather(gtf_v, [fidx])
            gy1 = plsc.load_gather(gtf_v, [fidx + 1])
            gx2 = plsc.load_gather(gtf_v, [fidx + 2])
            gy2 = plsc.load_gather(gtf_v, [fidx + 3])
            gcl = plsc.load_gather(gtf_v, [fidx + 8])

            pos = flag > 0
            regm0 = jnp.logical_and(pos, cls > 0.0)
            g = jnp.where(flag == 0, 0.0, gcl)
            g = jnp.where(flag < 0, -1.0, g)
            g = jnp.where(cls == 0.0, -1.0, g)
            clsm = g >= 0.0
            dx = gx1 - gx2
            dy = gy1 - gy2
            dxz = dx == 0.0
            dxs = jnp.where(dxz, 1.0, dx)
            slope = dy / dxs
            cy1 = jnp.where(dxz, 0.0, slope * (bx1 - gx2) + gy2)
            cy2 = jnp.where(dxz, 0.0, slope * (bx2 - gx2) + gy2)
            bw = bx2 - bx1
            bh = by2 - by1
            m = jnp.logical_and(jnp.logical_and(bw > 0.0, bh > 0.0),
                                jnp.logical_not(dxz))
            inv_bh = 1.0 / bh
            r1 = (cy1 - by2) * inv_bh
            r2 = (cy2 - by2) * inv_bh
            regm = jnp.logical_and(regm0, m)

            out_v[pl.ds(s, _L)] = g
            out_v[pl.ds(N + s, _L)] = jnp.where(clsm, 1.0, 0.0)
            out_v[pl.ds(2 * N + s, _L)] = r1
            out_v[pl.ds(3 * N + s, _L)] = r2
            out_v[pl.ds(4 * N + s, _L)] = jnp.where(regm, 1.0, 0.0)
            return carry

        lax.fori_loop(0, n_chunks, step, None)
        pltpu.sync_copy(out_v, out_hbm.at[w])

    return body(boxes2, gtb2, gtf2, gid2, flag2)


@jax.jit
def kernel(boxes, gt_boxes, gt_flanks, match_pos_flag, match_gt_id):
    B, N, _ = boxes.shape
    G = gt_boxes.shape[1]

    out = _sc_encode(boxes.reshape(B, 4 * N),
                     gt_boxes.reshape(B, 5 * G),
                     gt_flanks.reshape(B, 9 * G),
                     match_gt_id,
                     match_pos_flag,
                     B, N, G)
    o = out.reshape(B, 5, N)
    gdls_cls = o[:, 0][..., None]
    cls_label_mask = (o[:, 1] != 0.0)[..., None]
    reg_label = jnp.stack([o[:, 2], o[:, 3]], axis=-1)
    reg_label_mask = jnp.broadcast_to((o[:, 4] != 0.0)[..., None],
                                      reg_label.shape)
    return gdls_cls, cls_label_mask, reg_label, reg_label_mask


# speedup vs baseline: 18.2764x; 1.0022x over previous
import functools

import jax
import jax.numpy as jnp
from jax import lax
from jax.experimental import pallas as pl
from jax.experimental.pallas import tpu as pltpu, tpu_sc as plsc

_L = 16


def _sc_encode(boxes2, gtb2, gtf2, gid2, flag2, B, N, G):
    info = plsc.get_sparse_core_info()
    NC, NS = info.num_cores, info.num_subcores
    assert NC * NS == B, (NC, NS, B)
    n_chunks = (N + _L - 1) // _L
    mesh = plsc.VectorSubcoreMesh(core_axis_name="c", subcore_axis_name="s")

    @functools.partial(
        pl.kernel,
        out_type=jax.ShapeDtypeStruct((B, 5 * N), jnp.float32),
        mesh=mesh,
        compiler_params=pltpu.CompilerParams(
            needs_layout_passes=False,
            skip_device_barrier=True,
            disable_bounds_checks=True,
            disable_semaphore_checks=True,
        ),
        scratch_types=[
            pltpu.VMEM((4 * N,), jnp.float32),
            pltpu.VMEM((N,), jnp.int32),
            pltpu.VMEM((N,), jnp.int32),
            pltpu.VMEM((5 * G,), jnp.float32),
            pltpu.VMEM((9 * G,), jnp.float32),
            pltpu.VMEM((5 * N,), jnp.float32),
            pltpu.SemaphoreType.DMA,
        ],
    )
    def body(boxes_hbm, gtb_hbm, gtf_hbm, gid_hbm, flag_hbm, out_hbm,
             box_v, gid_v, flag_v, gtb_v, gtf_v, out_v, sem):
        w = lax.axis_index("s") * NC + lax.axis_index("c")
        cps = [
            pltpu.async_copy(boxes_hbm.at[w], box_v, sem),
            pltpu.async_copy(gid_hbm.at[w], gid_v, sem),
            pltpu.async_copy(flag_hbm.at[w], flag_v, sem),
            pltpu.async_copy(gtb_hbm.at[w], gtb_v, sem),
            pltpu.async_copy(gtf_hbm.at[w], gtf_v, sem),
        ]
        for cp in cps:
            cp.wait()

        lane = lax.iota(jnp.int32, _L)

        def step(c, carry):
            s = jnp.minimum(c * _L, N - _L)
            gid = gid_v[pl.ds(s, _L)]
            flag = flag_v[pl.ds(s, _L)]
            bidx = (s + lane) * 4
            bx1 = plsc.load_gather(box_v, [bidx])
            by1 = plsc.load_gather(box_v, [bidx + 1])
            bx2 = plsc.load_gather(box_v, [bidx + 2])
            by2 = plsc.load_gather(box_v, [bidx + 3])
            cls = plsc.load_gather(gtb_v, [gid * 5 + 4])
            fidx = gid * 9
            gx1 = plsc.load_gather(gtf_v, [fidx])
            gy1 = plsc.load_gather(gtf_v, [fidx + 1])
            gx2 = plsc.load_gather(gtf_v, [fidx + 2])
            gy2 = plsc.load_gather(gtf_v, [fidx + 3])
            gcl = plsc.load_gather(gtf_v, [fidx + 8])

            pos = flag > 0
            regm0 = jnp.logical_and(pos, cls > 0.0)
            g = jnp.where(flag == 0, 0.0, gcl)
            g = jnp.where(flag < 0, -1.0, g)
            g = jnp.where(cls == 0.0, -1.0, g)
            clsm = g >= 0.0
            dx = gx1 - gx2
            dy = gy1 - gy2
            dxz = dx == 0.0
            dxs = jnp.where(dxz, 1.0, dx)
            slope = dy / dxs
            cy1 = jnp.where(dxz, 0.0, slope * (bx1 - gx2) + gy2)
            cy2 = jnp.where(dxz, 0.0, slope * (bx2 - gx2) + gy2)
            bw = bx2 - bx1
            bh = by2 - by1
            m = jnp.logical_and(jnp.logical_and(bw > 0.0, bh > 0.0),
                                jnp.logical_not(dxz))
            inv_bh = 1.0 / bh
            r1 = (cy1 - by2) * inv_bh
            r2 = (cy2 - by2) * inv_bh
            regm = jnp.logical_and(regm0, m)

            out_v[pl.ds(s, _L)] = g
            out_v[pl.ds(N + s, _L)] = jnp.where(clsm, 1.0, 0.0)
            out_v[pl.ds(2 * N + s, _L)] = r1
            out_v[pl.ds(3 * N + s, _L)] = r2
            out_v[pl.ds(4 * N + s, _L)] = jnp.where(regm, 1.0, 0.0)
            return carry

        lax.fori_loop(0, n_chunks, step, None)
        pltpu.sync_copy(out_v, out_hbm.at[w])

    return body(boxes2, gtb2, gtf2, gid2, flag2)


@jax.jit
def kernel(boxes, gt_boxes, gt_flanks, match_pos_flag, match_gt_id):
    B, N, _ = boxes.shape
    G = gt_boxes.shape[1]

    out = _sc_encode(boxes.reshape(B, 4 * N),
                     gt_boxes.reshape(B, 5 * G),
                     gt_flanks.reshape(B, 9 * G),
                     match_gt_id,
                     match_pos_flag,
                     B, N, G)
    o = out.reshape(B, 5, N)
    gdls_cls = o[:, 0][..., None]
    cls_label_mask = (o[:, 1] != 0.0)[..., None]
    reg_label = jnp.stack([o[:, 2], o[:, 3]], axis=-1)
    reg_label_mask = jnp.broadcast_to((o[:, 4] != 0.0)[..., None],
                                      reg_label.shape)
    return gdls_cls, cls_label_mask, reg_label, reg_label_mask
